# baseline (speedup 1.0000x reference)
import functools

import jax
import jax.numpy as jnp
from jax import lax
from jax.experimental import pallas as pl
from jax.experimental.pallas import tpu as pltpu
from jax.experimental.pallas import tpu_sc as plsc

_CHUNK = 128


def _sc_gather3(user3, item13, item23, tu8, t18, t28, B, D, NC, NS):
    NW = NC * NS
    bpw = B // NW
    n_chunks = bpw // _CHUNK
    pack = 128 // D
    shift = pack.bit_length() - 1
    mesh = plsc.VectorSubcoreMesh(core_axis_name="c", subcore_axis_name="s")

    @functools.partial(
        pl.kernel,
        out_type=[jax.ShapeDtypeStruct((B // pack, 128), jnp.float32)] * 3,
        mesh=mesh,
        scratch_types=[pltpu.VMEM((n_chunks, _CHUNK), jnp.int32)] * 6
        + [pltpu.VMEM((bpw // pack, 128), jnp.float32)] * 3
        + [pltpu.VMEM((_CHUNK, 128), jnp.float32)] * 2
        + [pltpu.SemaphoreType.DMA] * 2,
        compiler_params=pltpu.CompilerParams(needs_layout_passes=False),
    )
    def k(u_hbm, i1_hbm, i2_hbm, tu_hbm, t1_hbm, t2_hbm,
          uo, o1, o2, uix, ix1, ix2, gu, g1, g2, ur, r1, r2,
          buf0, buf1, sem0, sem1):
        wid = lax.axis_index("s") * NC + lax.axis_index("c")
        base = wid * bpw
        idxs = (uix, ix1, ix2)
        gidxs = (gu, g1, g2)
        outs_v = (ur, r1, r2)
        outs_h = (uo, o1, o2)
        tbls = (tu_hbm, t1_hbm, t2_hbm)
        for src, ix in zip((u_hbm, i1_hbm, i2_hbm), idxs):
            pltpu.sync_copy(src.at[wid], ix)
        for ix, gx in zip(idxs, gidxs):
            for c in range(n_chunks):
                for v in range(_CHUNK // 16):
                    gx[c, pl.ds(v * 16, 16)] = lax.shift_right_logical(
                        ix[c, pl.ds(v * 16, 16)], shift)

        jobs = [(t, c) for t in range(3) for c in range(n_chunks)]
        bufs = (buf0, buf1)
        sems = (sem0, sem1)

        def fire(j):
            t, c = jobs[j]
            return pltpu.async_copy(
                tbls[t].at[gidxs[t].at[c]], bufs[j % 2], sems[j % 2])

        def extract(j):
            t, c = jobs[j]
            buf = bufs[j % 2]
            iota = lax.iota(jnp.int32, 16)

            def body(v, _):
                raw = idxs[t][c, pl.ds(v * 16, 16)]
                sub = (raw & (pack - 1)) * D
                rows = v * 16 + iota
                orow = c * _CHUNK + rows
                srow = lax.shift_right_logical(orow, shift)
                scol = (orow & (pack - 1)) * D
                for d in range(D):
                    val = plsc.load_gather(buf, [rows, sub + d])
                    plsc.store_scatter(outs_v[t], [srow, scol + d], val)
                return 0

            lax.fori_loop(0, _CHUNK // 16, body, 0)

        dsc = fire(0)
        for j in range(len(jobs)):
            dsc.wait()
            if j + 1 < len(jobs):
                nxt = fire(j + 1)
            extract(j)
            if j + 1 < len(jobs):
                dsc = nxt
        for rv, oh in zip(outs_v, outs_h):
            pltpu.sync_copy(rv, oh.at[pl.ds(wid * (bpw // pack), bpw // pack)])

    return k(user3, item13, item23, tu8, t18, t28)


def _tc_mlp_body(u_ref, i1_ref, i2_ref, w10, b10, w11, b11, w20, b20, w21, b21,
                 o1_ref, o2_ref):
    u = u_ref[0]
    e1 = u * i1_ref[0]
    e2 = u * i2_ref[0]
    h1 = jnp.maximum(
        jnp.dot(e1, w10[...], preferred_element_type=jnp.float32) + b10[...], 0.0)
    h2 = jnp.maximum(
        jnp.dot(e2, w20[...], preferred_element_type=jnp.float32) + b20[...], 0.0)
    s1 = jnp.sum(h1 * w11[...], axis=1, keepdims=True) + b11[...]
    s2 = jnp.sum(h2 * w21[...], axis=1, keepdims=True) + b21[...]
    o1_ref[0, 0] = jnp.maximum(s1, 0.0)[:, 0]
    o2_ref[0, 0] = jnp.maximum(s2, 0.0)[:, 0]


def _tc_mlp(u_e, i1_e, i2_e, W1_0, b1_0, W1_1, b1_1, W2_0, b2_0, W2_1, b2_1):
    B, D = u_e.shape
    NB = 8
    BLK = B // NB
    H = W1_0.shape[1]
    row = lambda i: (i, 0, 0)
    fixed2 = lambda i: (0, 0)
    in_specs = [
        pl.BlockSpec((1, BLK, D), row),
        pl.BlockSpec((1, BLK, D), row),
        pl.BlockSpec((1, BLK, D), row),
        pl.BlockSpec((D, H), fixed2),
        pl.BlockSpec((1, H), fixed2),
        pl.BlockSpec((1, H), fixed2),
        pl.BlockSpec((1, 1), fixed2),
        pl.BlockSpec((D, H), fixed2),
        pl.BlockSpec((1, H), fixed2),
        pl.BlockSpec((1, H), fixed2),
        pl.BlockSpec((1, 1), fixed2),
    ]
    out_specs = [
        pl.BlockSpec((1, 1, BLK), row),
        pl.BlockSpec((1, 1, BLK), row),
    ]
    o1, o2 = pl.pallas_call(
        _tc_mlp_body,
        grid=(NB,),
        in_specs=in_specs,
        out_specs=out_specs,
        out_shape=[jax.ShapeDtypeStruct((NB, 1, BLK), jnp.float32)] * 2,
    )(
        u_e.reshape(NB, BLK, D),
        i1_e.reshape(NB, BLK, D),
        i2_e.reshape(NB, BLK, D),
        W1_0, b1_0.reshape(1, H), W1_1.reshape(1, H), b1_1.reshape(1, 1),
        W2_0, b2_0.reshape(1, H), W2_1.reshape(1, H), b2_1.reshape(1, 1),
    )
    return o1.reshape(B), o2.reshape(B)


def kernel(user, item1, item2, U_MF, I_MF_1, I_MF_2,
           W1_0, b1_0, W1_1, b1_1, W2_0, b2_0, W2_1, b2_1):
    B = user.shape[0]
    D = U_MF.shape[1]
    info = plsc.get_sparse_core_info()
    NC, NS = info.num_cores, info.num_subcores
    NW = NC * NS
    n_chunks = (B // NW) // _CHUNK
    shp = (NW, n_chunks, _CHUNK)
    u3 = user.astype(jnp.int32).reshape(shp)
    it13 = item1.astype(jnp.int32).reshape(shp)
    it23 = item2.astype(jnp.int32).reshape(shp)
    V = U_MF.shape[0]
    pack = 128 // D
    u_p, i1_p, i2_p = _sc_gather3(
        u3, it13, it23,
        U_MF.reshape(V // pack, 128),
        I_MF_1.reshape(V // pack, 128),
        I_MF_2.reshape(V // pack, 128),
        B, D, NC, NS)
    u_e = u_p.reshape(B, D)
    i1_e = i1_p.reshape(B, D)
    i2_e = i2_p.reshape(B, D)
    o1, o2 = _tc_mlp(u_e, i1_e, i2_e,
                     W1_0, b1_0, W1_1, b1_1, W2_0, b2_0, W2_1, b2_1)
    return o1, o2, u_e, i1_e, i2_e


# speedup vs baseline: 1.2420x; 1.2420x over previous
import functools

import jax
import jax.numpy as jnp
from jax import lax
from jax.experimental import pallas as pl
from jax.experimental.pallas import tpu as pltpu
from jax.experimental.pallas import tpu_sc as plsc

_CHUNK = 16


def _sc_gather3(user3, item13, item23, tu8, t18, t28, B, D, NC, NS):
    NW = NC * NS
    bpw = B // NW
    n_groups = bpw // 16
    pack = 128 // D
    shift = pack.bit_length() - 1
    mesh = plsc.VectorSubcoreMesh(core_axis_name="c", subcore_axis_name="s")

    @functools.partial(
        pl.kernel,
        out_type=[jax.ShapeDtypeStruct((B // pack, 128), jnp.float32)] * 3,
        mesh=mesh,
        scratch_types=[pltpu.VMEM((n_groups, 16), jnp.int32)] * 3
        + [pltpu.VMEM((32, pack, D), jnp.float32)]
        + [pltpu.VMEM((bpw // pack, 128), jnp.float32)] * 3
        + [pltpu.SemaphoreType.DMA],
        compiler_params=pltpu.CompilerParams(needs_layout_passes=False),
    )
    def k(u_hbm, i1_hbm, i2_hbm, tu_hbm, t1_hbm, t2_hbm,
          uo, o1, o2, uix, ix1, ix2, buf, su, s1, s2, sem):
        wid = lax.axis_index("s") * NC + lax.axis_index("c")
        idxs = (uix, ix1, ix2)
        stgs = (su, s1, s2)
        outs_h = (uo, o1, o2)
        V = tu_hbm.shape[0]
        tbls = tuple(t.reshape(V // pack, pack, D)
                     for t in (tu_hbm, t1_hbm, t2_hbm))
        for src, ix in zip((u_hbm, i1_hbm, i2_hbm), idxs):
            pltpu.sync_copy(src.at[wid], ix)

        iota = lax.iota(jnp.int32, 16)

        def fire(t, g, ping):
            vec = idxs[t][g]
            slabs = lax.shift_right_logical(vec, shift)
            for l in range(16):
                pltpu.async_copy(
                    tbls[t].at[pl.ds(slabs[l], 1)],
                    buf.at[pl.ds(ping * 16 + l, 1)], sem)

        def drain16():
            for l in range(16):
                pltpu.make_async_copy(
                    tbls[0].at[pl.ds(0, 1)], buf.at[pl.ds(l, 1)], sem).wait()

        for t in range(3):
            fire(t, 0, 0)

            def body(g, _, t=t):
                ping = g & 1
                drain16()
                pl.when(g + 1 < n_groups)(
                    lambda: fire(t, g + 1, 1 - ping))
                vec = idxs[t][g]
                sub = vec & (pack - 1)
                rows = ping * 16 + iota
                orow = g * 16 + iota
                srow = lax.shift_right_logical(orow, shift)
                scol = (orow & (pack - 1)) * D
                for d in range(D):
                    val = plsc.load_gather(
                        buf, [rows, sub, jnp.full((16,), d, jnp.int32)])
                    plsc.store_scatter(stgs[t], [srow, scol + d], val)
                return 0

            lax.fori_loop(0, n_groups, body, 0)
        for stg, oh in zip(stgs, outs_h):
            pltpu.sync_copy(stg, oh.at[pl.ds(wid * (bpw // pack), bpw // pack)])

    return k(user3, item13, item23, tu8, t18, t28)


def _tc_mlp_body(u_ref, i1_ref, i2_ref, w10, b10, w11, b11, w20, b20, w21, b21,
                 o1_ref, o2_ref):
    u = u_ref[0]
    e1 = u * i1_ref[0]
    e2 = u * i2_ref[0]
    h1 = jnp.maximum(
        jnp.dot(e1, w10[...], preferred_element_type=jnp.float32) + b10[...], 0.0)
    h2 = jnp.maximum(
        jnp.dot(e2, w20[...], preferred_element_type=jnp.float32) + b20[...], 0.0)
    s1 = jnp.sum(h1 * w11[...], axis=1, keepdims=True) + b11[...]
    s2 = jnp.sum(h2 * w21[...], axis=1, keepdims=True) + b21[...]
    o1_ref[0, 0] = jnp.maximum(s1, 0.0)[:, 0]
    o2_ref[0, 0] = jnp.maximum(s2, 0.0)[:, 0]


def _tc_mlp(u_e, i1_e, i2_e, W1_0, b1_0, W1_1, b1_1, W2_0, b2_0, W2_1, b2_1):
    B, D = u_e.shape
    NB = 8
    BLK = B // NB
    H = W1_0.shape[1]
    row = lambda i: (i, 0, 0)
    fixed2 = lambda i: (0, 0)
    in_specs = [
        pl.BlockSpec((1, BLK, D), row),
        pl.BlockSpec((1, BLK, D), row),
        pl.BlockSpec((1, BLK, D), row),
        pl.BlockSpec((D, H), fixed2),
        pl.BlockSpec((1, H), fixed2),
        pl.BlockSpec((1, H), fixed2),
        pl.BlockSpec((1, 1), fixed2),
        pl.BlockSpec((D, H), fixed2),
        pl.BlockSpec((1, H), fixed2),
        pl.BlockSpec((1, H), fixed2),
        pl.BlockSpec((1, 1), fixed2),
    ]
    out_specs = [
        pl.BlockSpec((1, 1, BLK), row),
        pl.BlockSpec((1, 1, BLK), row),
    ]
    o1, o2 = pl.pallas_call(
        _tc_mlp_body,
        grid=(NB,),
        in_specs=in_specs,
        out_specs=out_specs,
        out_shape=[jax.ShapeDtypeStruct((NB, 1, BLK), jnp.float32)] * 2,
    )(
        u_e.reshape(NB, BLK, D),
        i1_e.reshape(NB, BLK, D),
        i2_e.reshape(NB, BLK, D),
        W1_0, b1_0.reshape(1, H), W1_1.reshape(1, H), b1_1.reshape(1, 1),
        W2_0, b2_0.reshape(1, H), W2_1.reshape(1, H), b2_1.reshape(1, 1),
    )
    return o1.reshape(B), o2.reshape(B)


def kernel(user, item1, item2, U_MF, I_MF_1, I_MF_2,
           W1_0, b1_0, W1_1, b1_1, W2_0, b2_0, W2_1, b2_1):
    B = user.shape[0]
    D = U_MF.shape[1]
    info = plsc.get_sparse_core_info()
    NC, NS = info.num_cores, info.num_subcores
    NW = NC * NS
    n_chunks = (B // NW) // _CHUNK
    shp = (NW, n_chunks, _CHUNK)
    u3 = user.astype(jnp.int32).reshape(shp)
    it13 = item1.astype(jnp.int32).reshape(shp)
    it23 = item2.astype(jnp.int32).reshape(shp)
    u_p, i1_p, i2_p = _sc_gather3(
        u3, it13, it23, U_MF, I_MF_1, I_MF_2, B, D, NC, NS)
    u_e = u_p.reshape(B, D)
    i1_e = i1_p.reshape(B, D)
    i2_e = i2_p.reshape(B, D)
    o1, o2 = _tc_mlp(u_e, i1_e, i2_e,
                     W1_0, b1_0, W1_1, b1_1, W2_0, b2_0, W2_1, b2_1)
    return o1, o2, u_e, i1_e, i2_e


# speedup vs baseline: 1.2878x; 1.0369x over previous
import functools

import jax
import jax.numpy as jnp
from jax import lax
from jax.experimental import pallas as pl
from jax.experimental.pallas import tpu as pltpu
from jax.experimental.pallas import tpu_sc as plsc

_CHUNK = 16


def _sc_gather3(user3, item13, item23, tu8, t18, t28, B, D, NC, NS):
    NW = NC * NS
    bpw = B // NW
    n_groups = bpw // 16
    pack = 128 // D
    shift = pack.bit_length() - 1
    mesh = plsc.VectorSubcoreMesh(core_axis_name="c", subcore_axis_name="s")

    @functools.partial(
        pl.kernel,
        out_type=[jax.ShapeDtypeStruct((B, D), jnp.float32)] * 3,
        mesh=mesh,
        scratch_types=[pltpu.VMEM((bpw,), jnp.int32)] * 3
        + [pltpu.VMEM((32, pack, D), jnp.float32)]
        + [pltpu.VMEM((2, 16, D), jnp.float32)] * 3
        + [pltpu.SemaphoreType.DMA] * 2,
        compiler_params=pltpu.CompilerParams(needs_layout_passes=False),
    )
    def k(u_hbm, i1_hbm, i2_hbm, tu_hbm, t1_hbm, t2_hbm,
          uo, o1, o2, uix, ix1, ix2, buf, su, s1, s2, sem, osem):
        wid = lax.axis_index("s") * NC + lax.axis_index("c")
        base = wid * bpw
        idxs = (uix, ix1, ix2)
        stgs = (su, s1, s2)
        outs_h = (uo, o1, o2)
        V = tu_hbm.shape[0]
        tbls = tuple(t.reshape(V // pack, pack, D)
                     for t in (tu_hbm, t1_hbm, t2_hbm))
        for src, ix in zip((u_hbm, i1_hbm, i2_hbm), idxs):
            pltpu.sync_copy(src.at[wid], ix)

        iota = lax.iota(jnp.int32, 16)

        def fire(t, g, ping):
            vec = idxs[t][pl.ds(g * 16, 16)]
            slabs = lax.shift_right_logical(vec, shift)
            for l in range(16):
                pltpu.async_copy(
                    tbls[t].at[pl.ds(slabs[l], 1)],
                    buf.at[pl.ds(ping * 16 + l, 1)], sem)

        def drain16():
            for l in range(16):
                pltpu.make_async_copy(
                    tbls[0].at[pl.ds(0, 1)], buf.at[pl.ds(l, 1)], sem).wait()

        def odrain(t):
            pltpu.make_async_copy(
                stgs[t].at[0], outs_h[t].at[pl.ds(0, 16)], osem).wait()

        for t in range(3):
            fire(t, 0, 0)

            def body(g, _, t=t):
                ping = g & 1
                drain16()
                pl.when(g + 1 < n_groups)(
                    lambda: fire(t, g + 1, 1 - ping))
                pl.when(g >= 2)(lambda: odrain(t))
                vec = idxs[t][pl.ds(g * 16, 16)]
                sub = vec & (pack - 1)
                rows = ping * 16 + iota
                pvec = jnp.zeros((16,), jnp.int32) + ping
                for d in range(D):
                    val = plsc.load_gather(
                        buf, [rows, sub, jnp.full((16,), d, jnp.int32)])
                    plsc.store_scatter(
                        stgs[t], [pvec, iota, jnp.full((16,), d, jnp.int32)],
                        val)
                pltpu.async_copy(
                    stgs[t].at[ping],
                    outs_h[t].at[pl.ds(base + g * 16, 16)], osem)
                return 0

            lax.fori_loop(0, n_groups, body, 0)
        for t in range(3):
            odrain(t)
            odrain(t)

    return k(user3, item13, item23, tu8, t18, t28)


def _tc_mlp_body(u_ref, i1_ref, i2_ref, w10, b10, w11, b11, w20, b20, w21, b21,
                 o1_ref, o2_ref):
    u = u_ref[0]
    e1 = u * i1_ref[0]
    e2 = u * i2_ref[0]
    h1 = jnp.maximum(
        jnp.dot(e1, w10[...], preferred_element_type=jnp.float32) + b10[...], 0.0)
    h2 = jnp.maximum(
        jnp.dot(e2, w20[...], preferred_element_type=jnp.float32) + b20[...], 0.0)
    s1 = jnp.sum(h1 * w11[...], axis=1, keepdims=True) + b11[...]
    s2 = jnp.sum(h2 * w21[...], axis=1, keepdims=True) + b21[...]
    o1_ref[0, 0] = jnp.maximum(s1, 0.0)[:, 0]
    o2_ref[0, 0] = jnp.maximum(s2, 0.0)[:, 0]


def _tc_mlp(u_e, i1_e, i2_e, W1_0, b1_0, W1_1, b1_1, W2_0, b2_0, W2_1, b2_1):
    B, D = u_e.shape
    NB = 8
    BLK = B // NB
    H = W1_0.shape[1]
    row = lambda i: (i, 0, 0)
    fixed2 = lambda i: (0, 0)
    in_specs = [
        pl.BlockSpec((1, BLK, D), row),
        pl.BlockSpec((1, BLK, D), row),
        pl.BlockSpec((1, BLK, D), row),
        pl.BlockSpec((D, H), fixed2),
        pl.BlockSpec((1, H), fixed2),
        pl.BlockSpec((1, H), fixed2),
        pl.BlockSpec((1, 1), fixed2),
        pl.BlockSpec((D, H), fixed2),
        pl.BlockSpec((1, H), fixed2),
        pl.BlockSpec((1, H), fixed2),
        pl.BlockSpec((1, 1), fixed2),
    ]
    out_specs = [
        pl.BlockSpec((1, 1, BLK), row),
        pl.BlockSpec((1, 1, BLK), row),
    ]
    o1, o2 = pl.pallas_call(
        _tc_mlp_body,
        grid=(NB,),
        in_specs=in_specs,
        out_specs=out_specs,
        out_shape=[jax.ShapeDtypeStruct((NB, 1, BLK), jnp.float32)] * 2,
    )(
        u_e.reshape(NB, BLK, D),
        i1_e.reshape(NB, BLK, D),
        i2_e.reshape(NB, BLK, D),
        W1_0, b1_0.reshape(1, H), W1_1.reshape(1, H), b1_1.reshape(1, 1),
        W2_0, b2_0.reshape(1, H), W2_1.reshape(1, H), b2_1.reshape(1, 1),
    )
    return o1.reshape(B), o2.reshape(B)


def kernel(user, item1, item2, U_MF, I_MF_1, I_MF_2,
           W1_0, b1_0, W1_1, b1_1, W2_0, b2_0, W2_1, b2_1):
    B = user.shape[0]
    D = U_MF.shape[1]
    info = plsc.get_sparse_core_info()
    NC, NS = info.num_cores, info.num_subcores
    NW = NC * NS
    shp = (NW, B // NW)
    u3 = user.astype(jnp.int32).reshape(shp)
    it13 = item1.astype(jnp.int32).reshape(shp)
    it23 = item2.astype(jnp.int32).reshape(shp)
    u_e, i1_e, i2_e = _sc_gather3(
        u3, it13, it23, U_MF, I_MF_1, I_MF_2, B, D, NC, NS)
    o1, o2 = _tc_mlp(u_e, i1_e, i2_e,
                     W1_0, b1_0, W1_1, b1_1, W2_0, b2_0, W2_1, b2_1)
    return o1, o2, u_e, i1_e, i2_e


# speedup vs baseline: 7.6937x; 5.9743x over previous
import functools

import jax
import jax.numpy as jnp
from jax import lax
from jax.experimental import pallas as pl
from jax.experimental.pallas import tpu as pltpu
from jax.experimental.pallas import tpu_sc as plsc

def _sc_gather3(user3, item13, item23, tuT, t1T, t2T, B, D, NC, NS):
    NW = NC * NS
    bpw = B // NW
    mesh = plsc.VectorSubcoreMesh(core_axis_name="c", subcore_axis_name="s")
    n_groups = bpw // 16

    @functools.partial(
        pl.kernel,
        out_type=[jax.ShapeDtypeStruct((D, B), jnp.float32)] * 3,
        mesh=mesh,
        scratch_types=[pltpu.VMEM((bpw,), jnp.int32)] * 3
        + [pltpu.VMEM((3, 16, D, 128), jnp.float32)]
        + [pltpu.VMEM((D, bpw), jnp.float32)] * 3
        + [pltpu.SemaphoreType.DMA],
        compiler_params=pltpu.CompilerParams(needs_layout_passes=False),
    )
    def k(u_hbm, i1_hbm, i2_hbm, tu_hbm, t1_hbm, t2_hbm,
          uo, o1, o2, uix, ix1, ix2, buf, vu, v1, v2, sem):
        wid = lax.axis_index("s") * NC + lax.axis_index("c")
        base = wid * bpw
        idxs = (uix, ix1, ix2)
        outs_v = (vu, v1, v2)
        outs_h = (uo, o1, o2)
        tbls = (tu_hbm, t1_hbm, t2_hbm)
        for src, ix in zip((u_hbm, i1_hbm, i2_hbm), idxs):
            pltpu.sync_copy(src.at[wid], ix)

        iota = lax.iota(jnp.int32, 16)

        def fire(t, g, pg):
            vec = idxs[t][pl.ds(g * 16, 16)]
            blks = vec & -128
            for l in range(16):
                pltpu.async_copy(
                    tbls[t].at[:, pl.ds(pl.multiple_of(blks[l], 128), 128)],
                    buf.at[pg, l], sem)

        def drain16(t):
            for l in range(16):
                pltpu.make_async_copy(
                    tbls[t].at[:, pl.ds(0, 128)], buf.at[0, l], sem).wait()

        for t in range(3):
            fire(t, 0, 0)
            fire(t, 1, 1)

            def body(g, _, t=t):
                ping = lax.rem(g, 3)
                drain16(t)
                pl.when(g + 2 < n_groups)(
                    lambda: fire(t, g + 2, lax.rem(g + 2, 3)))
                vec = idxs[t][pl.ds(g * 16, 16)]
                rmod = vec & 127
                pvec = jnp.zeros((16,), jnp.int32) + ping
                for d in range(D):
                    val = plsc.load_gather(
                        buf, [pvec, iota, jnp.full((16,), d, jnp.int32), rmod])
                    outs_v[t][d, pl.ds(g * 16, 16)] = val
                return 0

            lax.fori_loop(0, n_groups, body, 0)
        for rv, oh in zip(outs_v, outs_h):
            pltpu.sync_copy(rv, oh.at[:, pl.ds(base, bpw)])

    return k(user3, item13, item23, tuT, t1T, t2T)


def _tc_mlp_body(u_ref, i1_ref, i2_ref, w10t, b10c, w11t, b11, w20t, b20c,
                 w21t, b21, o1_ref, o2_ref):
    u = u_ref[...]
    e1 = u * i1_ref[...]
    e2 = u * i2_ref[...]
    h1 = jnp.maximum(
        jnp.dot(w10t[...], e1, preferred_element_type=jnp.float32)
        + b10c[...], 0.0)
    h2 = jnp.maximum(
        jnp.dot(w20t[...], e2, preferred_element_type=jnp.float32)
        + b20c[...], 0.0)
    s1 = jnp.dot(w11t[...], h1, preferred_element_type=jnp.float32) + b11[...]
    s2 = jnp.dot(w21t[...], h2, preferred_element_type=jnp.float32) + b21[...]
    o1_ref[0, 0] = jnp.maximum(s1, 0.0)[0]
    o2_ref[0, 0] = jnp.maximum(s2, 0.0)[0]


def _tc_mlp(uT, i1T, i2T, W1_0, b1_0, W1_1, b1_1, W2_0, b2_0, W2_1, b2_1):
    D, B = uT.shape
    NB = 2
    BLK = B // NB
    H = W1_0.shape[1]
    col = lambda i: (0, i)
    fixed2 = lambda i: (0, 0)
    in_specs = [
        pl.BlockSpec((D, BLK), col),
        pl.BlockSpec((D, BLK), col),
        pl.BlockSpec((D, BLK), col),
        pl.BlockSpec((H, D), fixed2),
        pl.BlockSpec((H, 1), fixed2),
        pl.BlockSpec((1, H), fixed2),
        pl.BlockSpec((1, 1), fixed2),
        pl.BlockSpec((H, D), fixed2),
        pl.BlockSpec((H, 1), fixed2),
        pl.BlockSpec((1, H), fixed2),
        pl.BlockSpec((1, 1), fixed2),
    ]
    out_specs = [
        pl.BlockSpec((1, 1, BLK), lambda i: (i, 0, 0)),
        pl.BlockSpec((1, 1, BLK), lambda i: (i, 0, 0)),
    ]
    o1, o2 = pl.pallas_call(
        _tc_mlp_body,
        grid=(NB,),
        in_specs=in_specs,
        out_specs=out_specs,
        out_shape=[jax.ShapeDtypeStruct((NB, 1, BLK), jnp.float32)] * 2,
    )(
        uT,
        i1T,
        i2T,
        W1_0.T, b1_0.reshape(H, 1), W1_1.T, b1_1.reshape(1, 1),
        W2_0.T, b2_0.reshape(H, 1), W2_1.T, b2_1.reshape(1, 1),
    )
    return o1.reshape(B), o2.reshape(B)


def kernel(user, item1, item2, U_MF, I_MF_1, I_MF_2,
           W1_0, b1_0, W1_1, b1_1, W2_0, b2_0, W2_1, b2_1):
    B = user.shape[0]
    D = U_MF.shape[1]
    info = plsc.get_sparse_core_info()
    NC, NS = info.num_cores, info.num_subcores
    NW = NC * NS
    shp = (NW, B // NW)
    u3 = user.astype(jnp.int32).reshape(shp)
    it13 = item1.astype(jnp.int32).reshape(shp)
    it23 = item2.astype(jnp.int32).reshape(shp)
    uT, i1T, i2T = _sc_gather3(
        u3, it13, it23, U_MF.T, I_MF_1.T, I_MF_2.T, B, D, NC, NS)
    o1, o2 = _tc_mlp(uT, i1T, i2T,
                     W1_0, b1_0, W1_1, b1_1, W2_0, b2_0, W2_1, b2_1)
    return o1, o2, uT.T, i1T.T, i2T.T
